# baseline (speedup 1.0000x reference)
import functools
import math

import jax
import jax.numpy as jnp
import numpy as np
from jax.experimental import pallas as pl
from jax.experimental.pallas import tpu as pltpu

HEADS = 16
HEAD_DIM = 128
HIDDEN = 2048
CHUNK = 16
TOKEN_BUDGET = 128
INIT_BUDGET = 128
RECENT_BUDGET = 128
THETA = 10000.0
NBLK = 4
CB = HIDDEN // NBLK


def _rope_tables(kv_len):
    inv_freq = 1.0 / (THETA ** (jnp.arange(0, HEAD_DIM, 2, dtype=jnp.float32) / HEAD_DIM))
    ang = jnp.float32(kv_len) * inv_freq
    cos = jnp.concatenate([jnp.cos(ang), jnp.cos(ang)])
    sin = jnp.concatenate([jnp.sin(ang), jnp.sin(ang)])
    cos_t = jnp.tile(cos, CB // HEAD_DIM)[None, :]
    sin_t = jnp.tile(sin, CB // HEAD_DIM)[None, :]
    r = np.zeros((HEAD_DIM, HEAD_DIM), np.float32)
    for j in range(64):
        r[j + 64, j] = -1.0
        r[j, j + 64] = 1.0
    rblk = np.kron(np.eye(CB // HEAD_DIM, dtype=np.float32), r)
    return cos_t, sin_t, jnp.asarray(rblk)


def _proj_kernel(h_ref, wq_ref, wk_ref, wv_ref, cos_ref, sin_ref, rot_ref,
                 q_ref, k_ref, v_ref):
    h = h_ref[...]
    q = jnp.dot(h, wq_ref[...], preferred_element_type=jnp.float32)
    k = jnp.dot(h, wk_ref[...], preferred_element_type=jnp.float32)
    v = jnp.dot(h, wv_ref[...], preferred_element_type=jnp.float32)
    cos = cos_ref[...]
    sin = sin_ref[...]
    rot = rot_ref[...]
    q_ref[...] = q * cos + jnp.dot(q, rot, preferred_element_type=jnp.float32, precision=jax.lax.Precision.HIGHEST) * sin
    k_ref[...] = k * cos + jnp.dot(k, rot, preferred_element_type=jnp.float32, precision=jax.lax.Precision.HIGHEST) * sin
    v_ref[...] = v


def _score_kernel(q_ref, k_ref, s_ref, ps_ref, *, kv_len, n_pages):
    q = q_ref[...].reshape(1, HEAD_DIM)
    k = k_ref[...].reshape(kv_len, HEAD_DIM)
    scale = jnp.float32(1.0 / math.sqrt(HEAD_DIM))
    s = jax.lax.dot_general(q, k, (((1,), (1,)), ((), ())),
                            preferred_element_type=jnp.float32, precision=jax.lax.Precision.HIGHEST)
    s_ref[...] = (s * scale).reshape(1, 1, kv_len)
    kp = k.reshape(n_pages, CHUNK, HEAD_DIM)
    kmin = kp.min(axis=1)
    kmax = kp.max(axis=1)
    m = jnp.maximum(kmin * q, kmax * q)
    ones = jnp.ones((1, HEAD_DIM), jnp.float32)
    ps_ref[...] = jax.lax.dot_general(ones, m, (((1,), (1,)), ((), ())),
                                      preferred_element_type=jnp.float32,
                                      precision=jax.lax.Precision.HIGHEST).reshape(1, 1, n_pages)


def _attend_kernel(s_ref, ps_ref, q_ref, kn_ref, vn_ref, v_ref, out_ref,
                   *, kv_len, n_pages, k_pages):
    s = s_ref[...].reshape(1, kv_len)
    ps = ps_ref[...].reshape(1, n_pages)
    q = q_ref[...].reshape(1, HEAD_DIM)
    kn = kn_ref[...].reshape(1, HEAD_DIM)
    scale = jnp.float32(1.0 / math.sqrt(HEAD_DIM))
    s_new = jnp.sum(q * kn) * scale
    lane_p = jax.lax.broadcasted_iota(jnp.int32, (1, n_pages), 1)
    lane_t = jax.lax.broadcasted_iota(jnp.int32, (1, kv_len), 1)
    mask = (lane_t < INIT_BUDGET) | (lane_t >= kv_len - RECENT_BUDGET)
    cur = ps
    neg = jnp.float32(-jnp.inf)
    for _ in range(k_pages):
        mval = jnp.max(cur)
        idx = jnp.min(jnp.where(cur == mval, lane_p, n_pages))
        mask = mask | ((lane_t >= idx * CHUNK) & (lane_t < idx * CHUNK + CHUNK))
        cur = jnp.where(lane_p == idx, neg, cur)
    sm = jnp.where(mask, s, jnp.float32(-1e9))
    mx = jnp.maximum(jnp.max(sm), s_new)
    e = jnp.where(mask, jnp.exp(sm - mx), jnp.float32(0.0))
    e_new = jnp.exp(s_new - mx)
    denom = jnp.sum(e) + e_new
    p = e / denom
    v = v_ref[...].reshape(kv_len, HEAD_DIM)
    out = jax.lax.dot_general(p, v, (((1,), (0,)), ((), ())),
                              preferred_element_type=jnp.float32, precision=jax.lax.Precision.HIGHEST)
    out_ref[...] = (out + (e_new / denom) * vn_ref[...].reshape(1, HEAD_DIM)
                    ).reshape(1, 1, HEAD_DIM)


def _outproj_kernel(x_ref, wo_ref, o_ref):
    o_ref[...] = jnp.dot(x_ref[...], wo_ref[...], preferred_element_type=jnp.float32, precision=jax.lax.Precision.HIGHEST)


def kernel(hidden_states, key_cache, value_cache, Wq, Wk, Wv, Wo):
    B, Q, _ = hidden_states.shape
    KV = key_cache.shape[2]
    BH = B * HEADS
    n_pages = KV // CHUNK
    k_pages = TOKEN_BUDGET // CHUNK

    h2 = hidden_states.reshape(B, HIDDEN)
    cos_t, sin_t, rblk = _rope_tables(KV)

    wspec = pl.BlockSpec((HIDDEN, CB), lambda j: (0, j))
    vecspec = pl.BlockSpec((1, CB), lambda j: (0, 0))
    q2, k2, v2 = pl.pallas_call(
        _proj_kernel,
        grid=(NBLK,),
        in_specs=[pl.BlockSpec((B, HIDDEN), lambda j: (0, 0)),
                  wspec, wspec, wspec, vecspec, vecspec,
                  pl.BlockSpec((CB, CB), lambda j: (0, 0))],
        out_specs=[pl.BlockSpec((B, CB), lambda j: (0, j))] * 3,
        out_shape=[jax.ShapeDtypeStruct((B, HIDDEN), jnp.float32)] * 3,
    )(h2, Wq, Wk, Wv, cos_t, sin_t, rblk)

    qf = q2.reshape(BH, 1, HEAD_DIM)
    knf = k2.reshape(BH, 1, HEAD_DIM)
    vnf = v2.reshape(BH, 1, HEAD_DIM)
    kc = key_cache.reshape(BH, KV, HEAD_DIM)
    vc = value_cache.reshape(BH, KV, HEAD_DIM)

    s, ps = pl.pallas_call(
        functools.partial(_score_kernel, kv_len=KV, n_pages=n_pages),
        grid=(BH,),
        in_specs=[pl.BlockSpec((1, 1, HEAD_DIM), lambda i: (i, 0, 0)),
                  pl.BlockSpec((1, KV, HEAD_DIM), lambda i: (i, 0, 0))],
        out_specs=[pl.BlockSpec((1, 1, KV), lambda i: (i, 0, 0)),
                   pl.BlockSpec((1, 1, n_pages), lambda i: (i, 0, 0))],
        out_shape=[jax.ShapeDtypeStruct((BH, 1, KV), jnp.float32),
                   jax.ShapeDtypeStruct((BH, 1, n_pages), jnp.float32)],
    )(qf, kc)

    out = pl.pallas_call(
        functools.partial(_attend_kernel, kv_len=KV, n_pages=n_pages,
                          k_pages=k_pages),
        grid=(BH,),
        in_specs=[pl.BlockSpec((1, 1, KV), lambda i: (i, 0, 0)),
                  pl.BlockSpec((1, 1, n_pages), lambda i: (i, 0, 0)),
                  pl.BlockSpec((1, 1, HEAD_DIM), lambda i: (i, 0, 0)),
                  pl.BlockSpec((1, 1, HEAD_DIM), lambda i: (i, 0, 0)),
                  pl.BlockSpec((1, 1, HEAD_DIM), lambda i: (i, 0, 0)),
                  pl.BlockSpec((1, KV, HEAD_DIM), lambda i: (i, 0, 0))],
        out_specs=pl.BlockSpec((1, 1, HEAD_DIM), lambda i: (i, 0, 0)),
        out_shape=jax.ShapeDtypeStruct((BH, 1, HEAD_DIM), jnp.float32),
    )(s, ps, qf, knf, vnf, vc)

    x = out.reshape(B, HIDDEN)
    res = pl.pallas_call(
        _outproj_kernel,
        grid=(NBLK,),
        in_specs=[pl.BlockSpec((B, HIDDEN), lambda j: (0, 0)),
                  pl.BlockSpec((HIDDEN, CB), lambda j: (0, j))],
        out_specs=pl.BlockSpec((B, CB), lambda j: (0, j)),
        out_shape=jax.ShapeDtypeStruct((B, HIDDEN), jnp.float32),
    )(x, Wo)
    return res.reshape(B, Q, HIDDEN)


# speedup vs baseline: 1.1857x; 1.1857x over previous
import functools
import math

import jax
import jax.numpy as jnp
import numpy as np
from jax import lax
from jax.experimental import pallas as pl
from jax.experimental.pallas import tpu as pltpu
from jax.experimental.pallas import tpu_sc as plsc

HEADS = 16
HEAD_DIM = 128
HIDDEN = 2048
CHUNK = 16
TOKEN_BUDGET = 128
INIT_BUDGET = 128
RECENT_BUDGET = 128
THETA = 10000.0
NBLK = 4
CB = HIDDEN // NBLK
HI = jax.lax.Precision.HIGHEST


def _rope_tables(kv_len):
    inv_freq = 1.0 / (THETA ** (jnp.arange(0, HEAD_DIM, 2, dtype=jnp.float32) / HEAD_DIM))
    ang = jnp.float32(kv_len) * inv_freq
    cos = jnp.concatenate([jnp.cos(ang), jnp.cos(ang)])
    sin = jnp.concatenate([jnp.sin(ang), jnp.sin(ang)])
    cos_t = jnp.tile(cos, CB // HEAD_DIM)[None, :]
    sin_t = jnp.tile(sin, CB // HEAD_DIM)[None, :]
    r = np.zeros((HEAD_DIM, HEAD_DIM), np.float32)
    for j in range(64):
        r[j + 64, j] = -1.0
        r[j, j + 64] = 1.0
    rblk = np.kron(np.eye(CB // HEAD_DIM, dtype=np.float32), r)
    return cos_t, sin_t, jnp.asarray(rblk)


def _proj_kernel(h_ref, wq_ref, wk_ref, wv_ref, cos_ref, sin_ref, rot_ref,
                 q_ref, k_ref, v_ref):
    h = h_ref[...]
    q = jnp.dot(h, wq_ref[...], preferred_element_type=jnp.float32)
    k = jnp.dot(h, wk_ref[...], preferred_element_type=jnp.float32)
    v = jnp.dot(h, wv_ref[...], preferred_element_type=jnp.float32)
    cos = cos_ref[...]
    sin = sin_ref[...]
    rot = rot_ref[...]
    q_ref[...] = q * cos + jnp.dot(q, rot, preferred_element_type=jnp.float32,
                                   precision=HI) * sin
    k_ref[...] = k * cos + jnp.dot(k, rot, preferred_element_type=jnp.float32,
                                   precision=HI) * sin
    v_ref[...] = v


def _score_kernel(q_ref, k_ref, s_ref, ps_ref, *, kv_len, n_pages):
    q = q_ref[...].reshape(1, HEAD_DIM)
    k = k_ref[...].reshape(kv_len, HEAD_DIM)
    scale = jnp.float32(1.0 / math.sqrt(HEAD_DIM))
    s = jax.lax.dot_general(q, k, (((1,), (1,)), ((), ())),
                            preferred_element_type=jnp.float32, precision=HI)
    s_ref[...] = (s * scale).reshape(1, 1, kv_len)
    kp = k.reshape(n_pages, 2, 8, HEAD_DIM)
    a = jnp.minimum(kp[:, 0], kp[:, 1])
    b = jnp.maximum(kp[:, 0], kp[:, 1])
    a = jnp.minimum(a[:, :4], a[:, 4:])
    b = jnp.maximum(b[:, :4], b[:, 4:])
    a = jnp.minimum(a[:, :2], a[:, 2:])
    b = jnp.maximum(b[:, :2], b[:, 2:])
    kmin = jnp.minimum(a[:, 0], a[:, 1])
    kmax = jnp.maximum(b[:, 0], b[:, 1])
    qpos = jnp.maximum(q, 0.0)
    qneg = jnp.minimum(q, 0.0)
    ps = (jax.lax.dot_general(qpos, kmax, (((1,), (1,)), ((), ())),
                              preferred_element_type=jnp.float32, precision=HI)
          + jax.lax.dot_general(qneg, kmin, (((1,), (1,)), ((), ())),
                                preferred_element_type=jnp.float32, precision=HI))
    ps_ref[...] = ps.reshape(1, 1, n_pages)


def _select_kernel(s_ref, ps_ref, q_ref, kn_ref, vn_ref,
                   p_ref, gid_ref, wm_ref, extra_ref,
                   *, kv_len, n_pages, k_pages, n_rows):
    s = s_ref[...]
    ps = ps_ref[...]
    q = q_ref[...]
    kn = kn_ref[...]
    scale = jnp.float32(1.0 / math.sqrt(HEAD_DIM))
    s_new = jnp.sum(q * kn, axis=1, keepdims=True) * scale
    lane_p = jax.lax.broadcasted_iota(jnp.int32, (n_rows, n_pages), 1)
    lane_t = jax.lax.broadcasted_iota(jnp.int32, (n_rows, kv_len), 1)
    lane16 = jax.lax.broadcasted_iota(jnp.int32, (n_rows, 16), 1)
    row16 = jax.lax.broadcasted_iota(jnp.int32, (n_rows, 16), 0)
    mask = (lane_t < INIT_BUDGET) | (lane_t >= kv_len - RECENT_BUDGET)
    cur = ps
    ids = jnp.zeros((n_rows, 16), jnp.int32)
    wm = jnp.zeros((n_rows, 16), jnp.float32)
    neg = jnp.float32(-jnp.inf)
    lo = INIT_BUDGET // CHUNK
    hi_b = (kv_len - RECENT_BUDGET) // CHUNK
    for i in range(k_pages):
        mval = jnp.max(cur, axis=1, keepdims=True)
        idx = jnp.min(jnp.where(cur == mval, lane_p, n_pages),
                      axis=1, keepdims=True)
        mask = mask | ((lane_t >= idx * CHUNK) & (lane_t < idx * CHUNK + CHUNK))
        cur = jnp.where(lane_p == idx, neg, cur)
        ids = jnp.where(lane16 == i, idx, ids)
        valid = (idx >= lo) & (idx < hi_b)
        wm = jnp.where((lane16 == i) & valid, jnp.float32(1.0), wm)
    sm = jnp.where(mask, s, jnp.float32(-1e9))
    mx = jnp.maximum(jnp.max(sm, axis=1, keepdims=True), s_new)
    e = jnp.where(mask, jnp.exp(sm - mx), jnp.float32(0.0))
    e_new = jnp.exp(s_new - mx)
    denom = jnp.sum(e, axis=1, keepdims=True) + e_new
    p_ref[...] = e / denom
    gid_ref[...] = ids + row16 * n_pages
    wm_ref[...] = wm
    extra_ref[...] = (e_new / denom) * vn_ref[...]


def _edge_kernel(p_ref, vi_ref, vr_ref, extra_ref, out_ref, *, kv_len):
    pi = p_ref[:, :, 0:INIT_BUDGET].reshape(1, INIT_BUDGET)
    pr = p_ref[:, :, kv_len - RECENT_BUDGET:kv_len].reshape(1, RECENT_BUDGET)
    vi = vi_ref[...].reshape(INIT_BUDGET, HEAD_DIM)
    vr = vr_ref[...].reshape(RECENT_BUDGET, HEAD_DIM)
    out = (jax.lax.dot_general(pi, vi, (((1,), (0,)), ((), ())),
                               preferred_element_type=jnp.float32)
           + jax.lax.dot_general(pr, vr, (((1,), (0,)), ((), ())),
                                 preferred_element_type=jnp.float32))
    out_ref[...] = (out + extra_ref[...].reshape(1, HEAD_DIM)).reshape(1, 1, HEAD_DIM)


def _outproj_kernel(x1_ref, x2_ref, wo_ref, o_ref):
    x = x1_ref[...] + x2_ref[...]
    o_ref[...] = jnp.dot(x, wo_ref[...], preferred_element_type=jnp.float32)


def _sc_combine(gid_ref, wm_ref, p_hbm, vp_hbm, out_ref,
                idx_v, wm_v, prow_v, rows_v, acc_v, sem,
                *, bh, k_pages, n_pages):
    c = lax.axis_index("c")
    s = lax.axis_index("s")
    wid = s * 2 + c
    per_w = bh // 32

    @pl.loop(0, per_w)
    def _row_loop(t):
        row = wid * per_w + t
        pltpu.sync_copy(gid_ref.at[row], idx_v)
        pltpu.sync_copy(wm_ref.at[row], wm_v)
        pltpu.sync_copy(p_hbm.at[row], prow_v)
        idx8 = idx_v.at[pl.ds(0, k_pages)]
        pltpu.async_copy(vp_hbm.at[idx8], rows_v, sem).wait()
        wvec = wm_v[...]
        ivec = idx_v[...] - row * n_pages
        acc = [jnp.zeros((16,), jnp.float32) for _ in range(8)]
        for j in range(k_pages):
            pid = ivec[j]
            ppw = prow_v[pl.ds(pid * CHUNK, CHUNK)] * wvec[j]
            for tt in range(CHUNK):
                pt = ppw[tt]
                for d in range(8):
                    acc[d] = acc[d] + pt * rows_v[j, pl.ds(tt * HEAD_DIM + d * 16, 16)]
        for d in range(8):
            acc_v[pl.ds(d * 16, 16)] = acc[d]
        pltpu.sync_copy(acc_v, out_ref.at[row])


def kernel(hidden_states, key_cache, value_cache, Wq, Wk, Wv, Wo):
    B, Q, _ = hidden_states.shape
    KV = key_cache.shape[2]
    BH = B * HEADS
    n_pages = KV // CHUNK
    k_pages = TOKEN_BUDGET // CHUNK

    h2 = hidden_states.reshape(B, HIDDEN)
    cos_t, sin_t, rblk = _rope_tables(KV)

    wspec = pl.BlockSpec((HIDDEN, CB), lambda j: (0, j))
    vecspec = pl.BlockSpec((1, CB), lambda j: (0, 0))
    q2, k2, v2 = pl.pallas_call(
        _proj_kernel,
        grid=(NBLK,),
        in_specs=[pl.BlockSpec((B, HIDDEN), lambda j: (0, 0)),
                  wspec, wspec, wspec, vecspec, vecspec,
                  pl.BlockSpec((CB, CB), lambda j: (0, 0))],
        out_specs=[pl.BlockSpec((B, CB), lambda j: (0, j))] * 3,
        out_shape=[jax.ShapeDtypeStruct((B, HIDDEN), jnp.float32)] * 3,
    )(h2, Wq, Wk, Wv, cos_t, sin_t, rblk)

    qf = q2.reshape(BH, 1, HEAD_DIM)
    kc = key_cache.reshape(BH, KV, HEAD_DIM)
    vc = value_cache.reshape(BH, KV, HEAD_DIM)

    s, ps = pl.pallas_call(
        functools.partial(_score_kernel, kv_len=KV, n_pages=n_pages),
        grid=(BH,),
        in_specs=[pl.BlockSpec((1, 1, HEAD_DIM), lambda i: (i, 0, 0)),
                  pl.BlockSpec((1, KV, HEAD_DIM), lambda i: (i, 0, 0))],
        out_specs=[pl.BlockSpec((1, 1, KV), lambda i: (i, 0, 0)),
                   pl.BlockSpec((1, 1, n_pages), lambda i: (i, 0, 0))],
        out_shape=[jax.ShapeDtypeStruct((BH, 1, KV), jnp.float32),
                   jax.ShapeDtypeStruct((BH, 1, n_pages), jnp.float32)],
    )(qf, kc)

    p, gid, wm, extra = pl.pallas_call(
        functools.partial(_select_kernel, kv_len=KV, n_pages=n_pages,
                          k_pages=k_pages, n_rows=BH),
        in_specs=[pl.BlockSpec((BH, KV), lambda: (0, 0)),
                  pl.BlockSpec((BH, n_pages), lambda: (0, 0)),
                  pl.BlockSpec((BH, HEAD_DIM), lambda: (0, 0)),
                  pl.BlockSpec((BH, HEAD_DIM), lambda: (0, 0)),
                  pl.BlockSpec((BH, HEAD_DIM), lambda: (0, 0))],
        out_specs=[pl.BlockSpec((BH, KV), lambda: (0, 0)),
                   pl.BlockSpec((BH, 16), lambda: (0, 0)),
                   pl.BlockSpec((BH, 16), lambda: (0, 0)),
                   pl.BlockSpec((BH, HEAD_DIM), lambda: (0, 0))],
        out_shape=[jax.ShapeDtypeStruct((BH, KV), jnp.float32),
                   jax.ShapeDtypeStruct((BH, 16), jnp.int32),
                   jax.ShapeDtypeStruct((BH, 16), jnp.float32),
                   jax.ShapeDtypeStruct((BH, HEAD_DIM), jnp.float32)],
    )(s.reshape(BH, KV), ps.reshape(BH, n_pages), q2.reshape(BH, HEAD_DIM),
      k2.reshape(BH, HEAD_DIM), v2.reshape(BH, HEAD_DIM))

    p3 = p.reshape(BH, 1, KV)
    out_edge = pl.pallas_call(
        functools.partial(_edge_kernel, kv_len=KV),
        grid=(BH,),
        in_specs=[pl.BlockSpec((1, 1, KV), lambda i: (i, 0, 0)),
                  pl.BlockSpec((1, INIT_BUDGET, HEAD_DIM), lambda i: (i, 0, 0)),
                  pl.BlockSpec((1, RECENT_BUDGET, HEAD_DIM),
                               lambda i: (i, (KV - RECENT_BUDGET) // RECENT_BUDGET, 0)),
                  pl.BlockSpec((1, 1, HEAD_DIM), lambda i: (i, 0, 0))],
        out_specs=pl.BlockSpec((1, 1, HEAD_DIM), lambda i: (i, 0, 0)),
        out_shape=jax.ShapeDtypeStruct((BH, 1, HEAD_DIM), jnp.float32),
    )(p3, vc, vc, extra.reshape(BH, 1, HEAD_DIM))

    vp = value_cache.reshape(BH * n_pages, CHUNK * HEAD_DIM)
    mesh = plsc.VectorSubcoreMesh(core_axis_name="c", subcore_axis_name="s")
    out_sc = pl.kernel(
        functools.partial(_sc_combine, bh=BH, k_pages=k_pages, n_pages=n_pages),
        mesh=mesh,
        out_type=jax.ShapeDtypeStruct((BH, HEAD_DIM), jnp.float32),
        scratch_types=[
            pltpu.VMEM((16,), jnp.int32),
            pltpu.VMEM((16,), jnp.float32),
            pltpu.VMEM((KV,), jnp.float32),
            pltpu.VMEM((k_pages, CHUNK * HEAD_DIM), jnp.float32),
            pltpu.VMEM((HEAD_DIM,), jnp.float32),
            pltpu.SemaphoreType.DMA,
        ],
    )(gid, wm, p, vp)

    x1 = out_edge.reshape(B, HIDDEN)
    x2 = out_sc.reshape(B, HIDDEN)
    res = pl.pallas_call(
        _outproj_kernel,
        grid=(NBLK,),
        in_specs=[pl.BlockSpec((B, HIDDEN), lambda j: (0, 0)),
                  pl.BlockSpec((B, HIDDEN), lambda j: (0, 0)),
                  pl.BlockSpec((HIDDEN, CB), lambda j: (0, j))],
        out_specs=pl.BlockSpec((B, CB), lambda j: (0, j)),
        out_shape=jax.ShapeDtypeStruct((B, HIDDEN), jnp.float32),
    )(x1, x2, Wo)
    return res.reshape(B, Q, HIDDEN)
